# baseline (speedup 1.0000x reference)
import jax
import jax.numpy as jnp
from jax.experimental import pallas as pl
from jax.experimental.pallas import tpu as pltpu

B, H, KV, DH = 16, 16, 4096, 128
BH = B * H


def _copy_patch_kernel(idx_ref, cur_ref, cache_ref, out_ref):
    out_ref[...] = cache_ref[...]
    bh_blk, kv_blk = out_ref.shape[0], out_ref.shape[1]
    i = pl.program_id(0)
    j = pl.program_id(1)
    slot = idx_ref[0] - 1
    off = slot - j * kv_blk

    @pl.when((off >= 0) & (off < kv_blk))
    def _():
        out_ref[:, pl.ds(off, 1), :] = cur_ref[pl.ds(i * bh_blk, bh_blk), :, :]


def kernel(cur, dim, idx, cache):
    del dim
    cache3 = cache.reshape(BH, KV, DH)
    cur3 = cur.reshape(BH, 1, DH)

    bh_blk = min(16, BH)
    kv_blk = min(1024, KV)
    grid = (BH // bh_blk, KV // kv_blk)

    out = pl.pallas_call(
        _copy_patch_kernel,
        grid=grid,
        in_specs=[
            pl.BlockSpec(memory_space=pltpu.SMEM),
            pl.BlockSpec((BH, 1, DH), lambda i, j: (0, 0, 0)),
            pl.BlockSpec((bh_blk, kv_blk, DH), lambda i, j: (i, j, 0)),
        ],
        out_specs=pl.BlockSpec((bh_blk, kv_blk, DH), lambda i, j: (i, j, 0)),
        out_shape=jax.ShapeDtypeStruct((BH, KV, DH), cache.dtype),
        compiler_params=pltpu.CompilerParams(
            dimension_semantics=("arbitrary", "arbitrary"),
            vmem_limit_bytes=63 * 1024 * 1024,
        ),
    )(idx, cur3, cache3)
    return out.reshape(B, H, KV, DH)


# speedup vs baseline: 1.0004x; 1.0004x over previous
import jax
import jax.numpy as jnp
from jax.experimental import pallas as pl
from jax.experimental.pallas import tpu as pltpu

B, H, KV, DH = 16, 16, 4096, 128
BH = B * H


def _copy_patch_kernel(idx_ref, cur_ref, cache_ref, out_ref):
    out_ref[...] = cache_ref[...]
    bh_blk, kv_blk = out_ref.shape[0], out_ref.shape[1]
    i = pl.program_id(0)
    j = pl.program_id(1)
    slot = idx_ref[0] - 1
    off = slot - j * kv_blk

    @pl.when((off >= 0) & (off < kv_blk))
    def _():
        out_ref[:, pl.ds(off, 1), :] = cur_ref[pl.ds(i * bh_blk, bh_blk), :, :]


def kernel(cur, dim, idx, cache):
    del dim
    cache3 = cache.reshape(BH, KV, DH)
    cur3 = cur.reshape(BH, 1, DH)

    bh_blk = min(8, BH)
    kv_blk = min(2048, KV)
    grid = (BH // bh_blk, KV // kv_blk)

    out = pl.pallas_call(
        _copy_patch_kernel,
        grid=grid,
        in_specs=[
            pl.BlockSpec(memory_space=pltpu.SMEM),
            pl.BlockSpec((BH, 1, DH), lambda i, j: (0, 0, 0)),
            pl.BlockSpec((bh_blk, kv_blk, DH), lambda i, j: (i, j, 0)),
        ],
        out_specs=pl.BlockSpec((bh_blk, kv_blk, DH), lambda i, j: (i, j, 0)),
        out_shape=jax.ShapeDtypeStruct((BH, KV, DH), cache.dtype),
        compiler_params=pltpu.CompilerParams(
            dimension_semantics=("arbitrary", "arbitrary"),
            vmem_limit_bytes=63 * 1024 * 1024,
        ),
    )(idx, cur3, cache3)
    return out.reshape(B, H, KV, DH)
